# baseline (speedup 1.0000x reference)
import functools

import jax
import jax.numpy as jnp
from jax import lax
from jax.experimental import pallas as pl
from jax.experimental.pallas import tpu as pltpu
from jax.experimental.pallas import tpu_sc as plsc

N = 10000
E = 320000
D = 128
C = 64
ED = 16

NP = 10240
NCORE = 2
NSUB = 16
NW = NCORE * NSUB
NPS = NP // NSUB
EPW = E // NW
EPWP = 10240
EP = NW * EPWP
ROWS = EP // 128
RPW = ROWS // NW
PAD_NODE = NP - 1

_MESH = plsc.VectorSubcoreMesh(core_axis_name="c", subcore_axis_name="s",
                               num_cores=NCORE, num_subcores=NSUB)

_SC_PARAMS = pltpu.CompilerParams(needs_layout_passes=False,
                                  use_tc_tiling_on_sc=False)

_HIGH = jax.lax.Precision.HIGHEST


def _dot(a, b):
    return jax.lax.dot_general(a, b, (((1,), (0,)), ((), ())),
                               precision=_HIGH,
                               preferred_element_type=jnp.float32)


def _lrelu(x):
    return jnp.maximum(x, 0.2 * x)


def _tc_node_pre(xpad, W1, as1, ad1):
    BN = 1024

    def body(x_ref, w_ref, as_ref, ad_ref, xp_ref, a_ref, d_ref):
        xp = _dot(x_ref[...], w_ref[...])
        xp_ref[...] = xp
        a_ref[...] = _dot(xp, as_ref[...])
        d_ref[...] = _dot(xp, ad_ref[...])

    return pl.pallas_call(
        body,
        grid=(NP // BN,),
        in_specs=[
            pl.BlockSpec((BN, D), lambda i: (i, 0)),
            pl.BlockSpec((D, C), lambda i: (0, 0)),
            pl.BlockSpec((C, 1), lambda i: (0, 0)),
            pl.BlockSpec((C, 1), lambda i: (0, 0)),
        ],
        out_specs=[
            pl.BlockSpec((BN, C), lambda i: (i, 0)),
            pl.BlockSpec((BN, 1), lambda i: (i, 0)),
            pl.BlockSpec((BN, 1), lambda i: (i, 0)),
        ],
        out_shape=[
            jax.ShapeDtypeStruct((NP, C), jnp.float32),
            jax.ShapeDtypeStruct((NP, 1), jnp.float32),
            jax.ShapeDtypeStruct((NP, 1), jnp.float32),
        ],
    )(xpad, W1, as1.reshape(C, 1), ad1.reshape(C, 1))


def _tc_edge(eaP, We1, ae1, We2, ae2, Wm1b, bm1):
    BE = 8192

    def body(ea_ref, we1_ref, ae1_ref, we2_ref, ae2_ref, wmb_ref, bm1_ref,
             g1_ref, g2_ref, r_ref):
        ea = ea_ref[...]
        v1 = _dot(we1_ref[...], ae1_ref[...])
        v2 = _dot(we2_ref[...], ae2_ref[...])
        g1_ref[...] = _dot(ea, v1)
        g2_ref[...] = _dot(ea, v2)
        r_ref[...] = _dot(ea, wmb_ref[...]) + bm1_ref[...]

    return pl.pallas_call(
        body,
        grid=(EP // BE,),
        in_specs=[
            pl.BlockSpec((BE, ED), lambda i: (i, 0)),
            pl.BlockSpec((ED, C), lambda i: (0, 0)),
            pl.BlockSpec((C, 1), lambda i: (0, 0)),
            pl.BlockSpec((ED, C), lambda i: (0, 0)),
            pl.BlockSpec((C, 1), lambda i: (0, 0)),
            pl.BlockSpec((ED, C), lambda i: (0, 0)),
            pl.BlockSpec((1, C), lambda i: (0, 0)),
        ],
        out_specs=[
            pl.BlockSpec((BE, 1), lambda i: (i, 0)),
            pl.BlockSpec((BE, 1), lambda i: (i, 0)),
            pl.BlockSpec((BE, C), lambda i: (i, 0)),
        ],
        out_shape=[
            jax.ShapeDtypeStruct((EP, 1), jnp.float32),
            jax.ShapeDtypeStruct((EP, 1), jnp.float32),
            jax.ShapeDtypeStruct((EP, C), jnp.float32),
        ],
    )(eaP, We1, ae1.reshape(C, 1), We2, ae2.reshape(C, 1), Wm1b,
      bm1.reshape(1, C))


def _combine(dn0, dn1, dg0, dg1, s0, s1, a_r, d_r, k_r, a0, a1, xp_r, b_r):
    deg = jnp.maximum(dg0[...] + dg1[...], 1.0)
    gs = (s0[...] + s1[...]) / deg
    al = _lrelu(a_r[...] + d_r[...] + gs)
    exs = jnp.exp(al - k_r[...])
    den = dn0[...] + dn1[...] + exs
    h = (a0[...] + a1[...] + exs * xp_r[...]) / den + b_r[...]
    return jnp.maximum(h, 0.0)


_NODE_SPECS = [
    pl.BlockSpec((1024, C), lambda i: (i, 0)),
    pl.BlockSpec((1024, C), lambda i: (i, 0)),
    pl.BlockSpec((1024, 1), lambda i: (i, 0)),
    pl.BlockSpec((1024, 1), lambda i: (i, 0)),
    pl.BlockSpec((1024, 1), lambda i: (i, 0)),
    pl.BlockSpec((1024, 1), lambda i: (i, 0)),
    pl.BlockSpec((1024, 1), lambda i: (i, 0)),
    pl.BlockSpec((1024, 1), lambda i: (i, 0)),
    pl.BlockSpec((1024, 1), lambda i: (i, 0)),
    pl.BlockSpec((1024, 1), lambda i: (i, 0)),
    pl.BlockSpec((1, 1), lambda i: (0, 0)),
    pl.BlockSpec((1024, C), lambda i: (i, 0)),
    pl.BlockSpec((1, C), lambda i: (0, 0)),
]


def _tc_node_layer(acc0, acc1, den0, den1, deg0, deg1, sg0, sg1, av, dv, kv,
                   xp, b, W, as_, ad_):
    BN = 1024

    def body(a0, a1, dn0, dn1, dg0, dg1, s0, s1, a_r, d_r, k_r, xp_r, b_r,
             w_ref, as_ref, ad_ref, xp2_ref, a_ref, d_ref):
        h = _combine(dn0, dn1, dg0, dg1, s0, s1, a_r, d_r, k_r, a0, a1,
                     xp_r, b_r)
        xp2 = _dot(h, w_ref[...])
        xp2_ref[...] = xp2
        a_ref[...] = _dot(xp2, as_ref[...])
        d_ref[...] = _dot(xp2, ad_ref[...])

    return pl.pallas_call(
        body,
        grid=(NP // BN,),
        in_specs=_NODE_SPECS + [
            pl.BlockSpec((C, C), lambda i: (0, 0)),
            pl.BlockSpec((C, 1), lambda i: (0, 0)),
            pl.BlockSpec((C, 1), lambda i: (0, 0)),
        ],
        out_specs=[
            pl.BlockSpec((BN, C), lambda i: (i, 0)),
            pl.BlockSpec((BN, 1), lambda i: (i, 0)),
            pl.BlockSpec((BN, 1), lambda i: (i, 0)),
        ],
        out_shape=[
            jax.ShapeDtypeStruct((NP, C), jnp.float32),
            jax.ShapeDtypeStruct((NP, 1), jnp.float32),
            jax.ShapeDtypeStruct((NP, 1), jnp.float32),
        ],
    )(acc0, acc1, den0, den1, deg0, deg1, sg0, sg1, av, dv, kv, xp,
      b.reshape(1, C), W, as_.reshape(C, 1), ad_.reshape(C, 1))


def _tc_node_mid(acc0, acc1, den0, den1, deg0, deg1, sg0, sg1, av, dv, kv,
                 xp, b, Wa, Wb):
    BN = 1024

    def body(a0, a1, dn0, dn1, dg0, dg1, s0, s1, a_r, d_r, k_r, xp_r, b_r,
             wa_ref, wb_ref, oa_ref, ob_ref):
        h = _combine(dn0, dn1, dg0, dg1, s0, s1, a_r, d_r, k_r, a0, a1,
                     xp_r, b_r)
        oa_ref[...] = _dot(h, wa_ref[...])
        ob_ref[...] = _dot(h, wb_ref[...])

    return pl.pallas_call(
        body,
        grid=(NP // BN,),
        in_specs=_NODE_SPECS + [
            pl.BlockSpec((C, C), lambda i: (0, 0)),
            pl.BlockSpec((C, C), lambda i: (0, 0)),
        ],
        out_specs=[
            pl.BlockSpec((BN, C), lambda i: (i, 0)),
            pl.BlockSpec((BN, C), lambda i: (i, 0)),
        ],
        out_shape=[
            jax.ShapeDtypeStruct((NP, C), jnp.float32),
            jax.ShapeDtypeStruct((NP, C), jnp.float32),
        ],
    )(acc0, acc1, den0, den1, deg0, deg1, sg0, sg1, av, dv, kv, xp,
      b.reshape(1, C), Wa, Wb)


def _sc_layer1(aT, dT, g1F, g2F, srcF, dstF, kv, xp):
    @functools.partial(
        pl.kernel,
        out_type=(
            jax.ShapeDtypeStruct((NCORE * NP, C), jnp.float32),
            jax.ShapeDtypeStruct((NCORE * NP,), jnp.float32),
            jax.ShapeDtypeStruct((NCORE * NP,), jnp.float32),
            jax.ShapeDtypeStruct((NCORE * NP,), jnp.float32),
            jax.ShapeDtypeStruct((NCORE * NP,), jnp.float32),
        ),
        mesh=_MESH,
        compiler_params=_SC_PARAMS,
        scratch_types=[
            pltpu.VMEM((NP,), jnp.float32),
            pltpu.VMEM((NP,), jnp.float32),
            pltpu.VMEM((16,), jnp.float32),
            pltpu.VMEM((EPWP,), jnp.int32),
            pltpu.VMEM((EPWP,), jnp.int32),
            pltpu.VMEM((EPWP,), jnp.float32),
            pltpu.VMEM((EPWP,), jnp.float32),
            pltpu.VMEM((EPWP,), jnp.float32),
            pltpu.VMEM((NPS,), jnp.float32),
            pltpu.VMEM((128, C), jnp.float32),
            pltpu.VMEM_SHARED((NP,), jnp.float32),
            pltpu.VMEM_SHARED((NP,), jnp.float32),
            pltpu.VMEM_SHARED((NP,), jnp.float32),
            pltpu.VMEM_SHARED((NP,), jnp.float32),
            pltpu.VMEM_SHARED((NP, C), jnp.float32),
        ],
    )
    def k(aT_h, dT_h, g1_h, g2_h, srcF_h, dstF_h, kv_h, xp_h,
          acc_o, den_o, deg_o, sg1_o, sg2_o,
          ta, td, kvv, srcf, dstf, g1v, g2v, exv, zv,
          rows, dens, degs, sg1s, sg2s, accs):
        c = lax.axis_index("c")
        s = lax.axis_index("s")
        w = c * NSUB + s

        @pl.loop(0, NPS, step=16)
        def _(i):
            zv[pl.ds(i, 16)] = jnp.zeros((16,), jnp.float32)

        @pl.loop(0, 128)
        def _(r):
            @pl.loop(0, C, step=16)
            def _(i):
                rows[r, pl.ds(i, 16)] = jnp.zeros((16,), jnp.float32)

        ns = s * NPS
        pltpu.sync_copy(zv, dens.at[pl.ds(ns, NPS)])
        pltpu.sync_copy(zv, degs.at[pl.ds(ns, NPS)])
        pltpu.sync_copy(zv, sg1s.at[pl.ds(ns, NPS)])
        pltpu.sync_copy(zv, sg2s.at[pl.ds(ns, NPS)])

        @pl.loop(0, NPS, step=128)
        def _(i):
            pltpu.sync_copy(rows, accs.at[pl.ds(ns + i, 128)])

        eb = w * EPWP
        esl = pl.ds(eb, EPWP)
        pltpu.sync_copy(aT_h, ta)
        pltpu.sync_copy(dT_h, td)
        pltpu.sync_copy(kv_h, kvv)
        pltpu.sync_copy(srcF_h.at[esl], srcf)
        pltpu.sync_copy(dstF_h.at[esl], dstf)
        pltpu.sync_copy(g1_h.at[esl], g1v)
        pltpu.sync_copy(g2_h.at[esl], g2v)
        plsc.subcore_barrier()
        K = kvv[...]

        @pl.loop(0, EPWP, step=16)
        def _(i):
            g = pl.ds(i, 16)
            av = plsc.load_gather(ta, [srcf[g]])
            dv = plsc.load_gather(td, [dstf[g]])
            al = _lrelu(av + dv + g1v[g])
            exv[g] = jnp.exp(al - K)

        pltpu.sync_copy(exv, dens.at[dstf], add=True)
        pltpu.sync_copy(g1v, sg1s.at[dstf], add=True)
        pltpu.sync_copy(g2v, sg2s.at[dstf], add=True)

        @pl.loop(0, EPWP, step=16)
        def _(i):
            g2v[pl.ds(i, 16)] = jnp.ones((16,), jnp.float32)

        pltpu.sync_copy(g2v, degs.at[dstf], add=True)

        @pl.loop(0, RPW)
        def _(j):
            pltpu.sync_copy(xp_h.at[srcf.at[pl.ds(j * 128, 128)]], rows)

            @pl.loop(0, 128)
            def _(e):
                esplat = jnp.zeros((16,), jnp.int32) + (j * 128 + e)
                we = plsc.load_gather(exv, [esplat])
                for cb in range(C // 16):
                    g = pl.ds(cb * 16, 16)
                    rows[e, g] = rows[e, g] * we

            pltpu.sync_copy(rows, accs.at[dstf.at[pl.ds(j * 128, 128)]],
                            add=True)

        plsc.subcore_barrier()
        sl = pl.ds(ns, NPS)
        osl = pl.ds(c * NP + ns, NPS)
        pltpu.sync_copy(accs.at[sl], acc_o.at[osl])
        pltpu.sync_copy(dens.at[sl], den_o.at[osl])
        pltpu.sync_copy(degs.at[sl], deg_o.at[osl])
        pltpu.sync_copy(sg1s.at[sl], sg1_o.at[osl])
        pltpu.sync_copy(sg2s.at[sl], sg2_o.at[osl])

    return k(aT, dT, g1F, g2F, srcF, dstF, kv, xp)


def _sc_layer2(aT, dT, gF, srcF, dstF, kv, xp):
    @functools.partial(
        pl.kernel,
        out_type=(
            jax.ShapeDtypeStruct((NCORE * NP, C), jnp.float32),
            jax.ShapeDtypeStruct((NCORE * NP,), jnp.float32),
        ),
        mesh=_MESH,
        compiler_params=_SC_PARAMS,
        scratch_types=[
            pltpu.VMEM((NP,), jnp.float32),
            pltpu.VMEM((NP,), jnp.float32),
            pltpu.VMEM((16,), jnp.float32),
            pltpu.VMEM((EPWP,), jnp.int32),
            pltpu.VMEM((EPWP,), jnp.int32),
            pltpu.VMEM((EPWP,), jnp.float32),
            pltpu.VMEM((EPWP,), jnp.float32),
            pltpu.VMEM((NPS,), jnp.float32),
            pltpu.VMEM((128, C), jnp.float32),
            pltpu.VMEM_SHARED((NP,), jnp.float32),
            pltpu.VMEM_SHARED((NP, C), jnp.float32),
        ],
    )
    def k(aT_h, dT_h, g_h, srcF_h, dstF_h, kv_h, xp_h,
          acc_o, den_o,
          ta, td, kvv, srcf, dstf, gv, exv, zv,
          rows, dens, accs):
        c = lax.axis_index("c")
        s = lax.axis_index("s")
        w = c * NSUB + s

        @pl.loop(0, NPS, step=16)
        def _(i):
            zv[pl.ds(i, 16)] = jnp.zeros((16,), jnp.float32)

        @pl.loop(0, 128)
        def _(r):
            @pl.loop(0, C, step=16)
            def _(i):
                rows[r, pl.ds(i, 16)] = jnp.zeros((16,), jnp.float32)

        ns = s * NPS
        pltpu.sync_copy(zv, dens.at[pl.ds(ns, NPS)])

        @pl.loop(0, NPS, step=128)
        def _(i):
            pltpu.sync_copy(rows, accs.at[pl.ds(ns + i, 128)])

        eb = w * EPWP
        esl = pl.ds(eb, EPWP)
        pltpu.sync_copy(aT_h, ta)
        pltpu.sync_copy(dT_h, td)
        pltpu.sync_copy(kv_h, kvv)
        pltpu.sync_copy(srcF_h.at[esl], srcf)
        pltpu.sync_copy(dstF_h.at[esl], dstf)
        pltpu.sync_copy(g_h.at[esl], gv)
        plsc.subcore_barrier()
        K = kvv[...]

        @pl.loop(0, EPWP, step=16)
        def _(i):
            g = pl.ds(i, 16)
            av = plsc.load_gather(ta, [srcf[g]])
            dv = plsc.load_gather(td, [dstf[g]])
            al = _lrelu(av + dv + gv[g])
            exv[g] = jnp.exp(al - K)

        pltpu.sync_copy(exv, dens.at[dstf], add=True)

        @pl.loop(0, RPW)
        def _(j):
            pltpu.sync_copy(xp_h.at[srcf.at[pl.ds(j * 128, 128)]], rows)

            @pl.loop(0, 128)
            def _(e):
                esplat = jnp.zeros((16,), jnp.int32) + (j * 128 + e)
                we = plsc.load_gather(exv, [esplat])
                for cb in range(C // 16):
                    g = pl.ds(cb * 16, 16)
                    rows[e, g] = rows[e, g] * we

            pltpu.sync_copy(rows, accs.at[dstf.at[pl.ds(j * 128, 128)]],
                            add=True)

        plsc.subcore_barrier()
        sl = pl.ds(ns, NPS)
        osl = pl.ds(c * NP + ns, NPS)
        pltpu.sync_copy(accs.at[sl], acc_o.at[osl])
        pltpu.sync_copy(dens.at[sl], den_o.at[osl])

    return k(aT, dT, gF, srcF, dstF, kv, xp)


def _sc_gather_pq(Pt, Qt, srcP, dstP):
    @functools.partial(
        pl.kernel,
        out_type=(
            jax.ShapeDtypeStruct((EP, C), jnp.float32),
            jax.ShapeDtypeStruct((EP, C), jnp.float32),
        ),
        mesh=_MESH,
        compiler_params=_SC_PARAMS,
        scratch_types=[
            pltpu.VMEM((RPW, 128), jnp.int32),
            pltpu.VMEM((RPW, 128), jnp.int32),
            pltpu.VMEM((128, C), jnp.float32),
            pltpu.VMEM((128, C), jnp.float32),
        ],
    )
    def k(p_h, q_h, src_h, dst_h,
          gp_o, gq_o,
          srcv, dstv, prow, qrow):
        c = lax.axis_index("c")
        s = lax.axis_index("s")
        w = c * NSUB + s
        row0 = w * RPW
        pltpu.sync_copy(src_h.at[pl.ds(row0, RPW)], srcv)
        pltpu.sync_copy(dst_h.at[pl.ds(row0, RPW)], dstv)

        @pl.loop(0, RPW)
        def _(j):
            e0 = (row0 + j) * 128
            pltpu.sync_copy(p_h.at[srcv.at[j]], prow)
            pltpu.sync_copy(prow, gp_o.at[pl.ds(e0, 128)])
            pltpu.sync_copy(q_h.at[dstv.at[j]], qrow)
            pltpu.sync_copy(qrow, gq_o.at[pl.ds(e0, 128)])

    return k(Pt, Qt, srcP, dstP)


def _tc_final(GP, GQ, RP, wm2, bm2):
    BE = 8192

    def body(gp_ref, gq_ref, r_ref, w_ref, b_ref, out_ref):
        h = jnp.maximum(gp_ref[...] + gq_ref[...] + r_ref[...], 0.0)
        out_ref[...] = _dot(h, w_ref[...]) + b_ref[...]

    return pl.pallas_call(
        body,
        grid=(EP // BE,),
        in_specs=[
            pl.BlockSpec((BE, C), lambda i: (i, 0)),
            pl.BlockSpec((BE, C), lambda i: (i, 0)),
            pl.BlockSpec((BE, C), lambda i: (i, 0)),
            pl.BlockSpec((C, 1), lambda i: (0, 0)),
            pl.BlockSpec((1, 1), lambda i: (0, 0)),
        ],
        out_specs=pl.BlockSpec((BE, 1), lambda i: (i, 0)),
        out_shape=jax.ShapeDtypeStruct((EP, 1), jnp.float32),
    )(GP, GQ, RP, wm2.reshape(C, 1), bm2.reshape(1, 1))


def kernel(x, edge_index, edge_attr, W1, as1, ad1, We1, ae1, b1,
           W2, as2, ad2, We2, ae2, b2, Wm1, bm1, Wm2, bm2):
    src = edge_index[0]
    dst = edge_index[1]

    xpad = jnp.pad(x, ((0, NP - N), (0, 0)))
    srcF = jnp.pad(src.reshape(NW, EPW), ((0, 0), (0, EPWP - EPW)),
                   constant_values=0).reshape(EP)
    dstF = jnp.pad(dst.reshape(NW, EPW), ((0, 0), (0, EPWP - EPW)),
                   constant_values=PAD_NODE).reshape(EP)
    srcP = srcF.reshape(ROWS, 128)
    dstP = dstF.reshape(ROWS, 128)
    eaP = jnp.pad(edge_attr.reshape(NW, EPW, ED), ((0, 0), (0, EPWP - EPW), (0, 0))
                  ).reshape(EP, ED)

    xp1, a1, d1 = _tc_node_pre(xpad, W1, as1, ad1)
    g1, g2, RP = _tc_edge(eaP, We1, ae1, We2, ae2, Wm1[C:C + ED], bm1)
    a1f = a1.reshape(NP)
    d1f = d1.reshape(NP)
    g1F = g1.reshape(EP)
    g2F = g2.reshape(EP)

    K1 = _lrelu(jnp.max(a1f) + jnp.max(d1f) + jnp.maximum(jnp.max(g1F), 0.0))
    kv1 = jnp.full((16,), K1, jnp.float32)

    accf1, den1p, degp, sg1p, sg2p = _sc_layer1(
        a1f, d1f, g1F, g2F, srcF, dstF, kv1, xp1)

    xp2, a2, d2 = _tc_node_layer(
        accf1[:NP], accf1[NP:],
        den1p[:NP].reshape(NP, 1), den1p[NP:].reshape(NP, 1),
        degp[:NP].reshape(NP, 1), degp[NP:].reshape(NP, 1),
        sg1p[:NP].reshape(NP, 1), sg1p[NP:].reshape(NP, 1),
        a1, d1, K1.reshape(1, 1), xp1, b1, W2, as2, ad2)
    a2f = a2.reshape(NP)
    d2f = d2.reshape(NP)
    K2 = _lrelu(jnp.max(a2f) + jnp.max(d2f) + jnp.maximum(jnp.max(g2F), 0.0))
    kv2 = jnp.full((16,), K2, jnp.float32)

    accf2, den2p = _sc_layer2(a2f, d2f, g2F, srcF, dstF, kv2, xp2)

    Pt, Qt = _tc_node_mid(
        accf2[:NP], accf2[NP:],
        den2p[:NP].reshape(NP, 1), den2p[NP:].reshape(NP, 1),
        degp[:NP].reshape(NP, 1), degp[NP:].reshape(NP, 1),
        sg2p[:NP].reshape(NP, 1), sg2p[NP:].reshape(NP, 1),
        a2, d2, K2.reshape(1, 1), xp2, b2, Wm1[:C], Wm1[C + ED:])

    GP, GQ = _sc_gather_pq(Pt, Qt, srcP, dstP)
    outP = _tc_final(GP, GQ, RP, Wm2.reshape(C), bm2)

    out = outP.reshape(NW, EPWP)[:, :EPW].reshape(E, 1)
    return out


# speedup vs baseline: 1.1649x; 1.1649x over previous
import functools

import jax
import jax.numpy as jnp
from jax import lax
from jax.experimental import pallas as pl
from jax.experimental.pallas import tpu as pltpu
from jax.experimental.pallas import tpu_sc as plsc

N = 10000
E = 320000
D = 128
C = 64
ED = 16

NP = 10240
NCORE = 2
NSUB = 16
NW = NCORE * NSUB
NPS = NP // NSUB
EPW = E // NW
EPWP = 10240
EP = NW * EPWP
ROWS = EP // 128
RPW = ROWS // NW
PAD_NODE = NP - 1

_MESH = plsc.VectorSubcoreMesh(core_axis_name="c", subcore_axis_name="s",
                               num_cores=NCORE, num_subcores=NSUB)

_SC_PARAMS = pltpu.CompilerParams(needs_layout_passes=False,
                                  use_tc_tiling_on_sc=False)

_HIGH = jax.lax.Precision.HIGHEST


def _dot(a, b):
    return jax.lax.dot_general(a, b, (((1,), (0,)), ((), ())),
                               precision=_HIGH,
                               preferred_element_type=jnp.float32)


def _lrelu(x):
    return jnp.maximum(x, 0.2 * x)


def _tc_node_pre(xpad, W1, as1, ad1):
    BN = 1024

    def body(x_ref, w_ref, as_ref, ad_ref, xp_ref, a_ref, d_ref):
        xp = _dot(x_ref[...], w_ref[...])
        xp_ref[...] = xp
        a_ref[...] = _dot(xp, as_ref[...])
        d_ref[...] = _dot(xp, ad_ref[...])

    return pl.pallas_call(
        body,
        grid=(NP // BN,),
        in_specs=[
            pl.BlockSpec((BN, D), lambda i: (i, 0)),
            pl.BlockSpec((D, C), lambda i: (0, 0)),
            pl.BlockSpec((C, 1), lambda i: (0, 0)),
            pl.BlockSpec((C, 1), lambda i: (0, 0)),
        ],
        out_specs=[
            pl.BlockSpec((BN, C), lambda i: (i, 0)),
            pl.BlockSpec((BN, 1), lambda i: (i, 0)),
            pl.BlockSpec((BN, 1), lambda i: (i, 0)),
        ],
        out_shape=[
            jax.ShapeDtypeStruct((NP, C), jnp.float32),
            jax.ShapeDtypeStruct((NP, 1), jnp.float32),
            jax.ShapeDtypeStruct((NP, 1), jnp.float32),
        ],
    )(xpad, W1, as1.reshape(C, 1), ad1.reshape(C, 1))


def _tc_edge(eaP, We1, ae1, We2, ae2, Wm1b, bm1):
    BE = 8192

    def body(ea_ref, we1_ref, ae1_ref, we2_ref, ae2_ref, wmb_ref, bm1_ref,
             g1_ref, g2_ref, r_ref):
        ea = ea_ref[...]
        v1 = _dot(we1_ref[...], ae1_ref[...])
        v2 = _dot(we2_ref[...], ae2_ref[...])
        g1_ref[...] = _dot(ea, v1)
        g2_ref[...] = _dot(ea, v2)
        r_ref[...] = _dot(ea, wmb_ref[...]) + bm1_ref[...]

    return pl.pallas_call(
        body,
        grid=(EP // BE,),
        in_specs=[
            pl.BlockSpec((BE, ED), lambda i: (i, 0)),
            pl.BlockSpec((ED, C), lambda i: (0, 0)),
            pl.BlockSpec((C, 1), lambda i: (0, 0)),
            pl.BlockSpec((ED, C), lambda i: (0, 0)),
            pl.BlockSpec((C, 1), lambda i: (0, 0)),
            pl.BlockSpec((ED, C), lambda i: (0, 0)),
            pl.BlockSpec((1, C), lambda i: (0, 0)),
        ],
        out_specs=[
            pl.BlockSpec((BE, 1), lambda i: (i, 0)),
            pl.BlockSpec((BE, 1), lambda i: (i, 0)),
            pl.BlockSpec((BE, C), lambda i: (i, 0)),
        ],
        out_shape=[
            jax.ShapeDtypeStruct((EP, 1), jnp.float32),
            jax.ShapeDtypeStruct((EP, 1), jnp.float32),
            jax.ShapeDtypeStruct((EP, C), jnp.float32),
        ],
    )(eaP, We1, ae1.reshape(C, 1), We2, ae2.reshape(C, 1), Wm1b,
      bm1.reshape(1, C))


def _combine(dn0, dn1, dg0, dg1, s0, s1, a_r, d_r, k_r, a0, a1, xp_r, b_r):
    deg = jnp.maximum(dg0[...] + dg1[...], 1.0)
    gs = (s0[...] + s1[...]) / deg
    al = _lrelu(a_r[...] + d_r[...] + gs)
    exs = jnp.exp(al - k_r[...])
    den = dn0[...] + dn1[...] + exs
    h = (a0[...] + a1[...] + exs * xp_r[...]) / den + b_r[...]
    return jnp.maximum(h, 0.0)


_NODE_SPECS = [
    pl.BlockSpec((1024, C), lambda i: (i, 0)),
    pl.BlockSpec((1024, C), lambda i: (i, 0)),
    pl.BlockSpec((1024, 1), lambda i: (i, 0)),
    pl.BlockSpec((1024, 1), lambda i: (i, 0)),
    pl.BlockSpec((1024, 1), lambda i: (i, 0)),
    pl.BlockSpec((1024, 1), lambda i: (i, 0)),
    pl.BlockSpec((1024, 1), lambda i: (i, 0)),
    pl.BlockSpec((1024, 1), lambda i: (i, 0)),
    pl.BlockSpec((1024, 1), lambda i: (i, 0)),
    pl.BlockSpec((1024, 1), lambda i: (i, 0)),
    pl.BlockSpec((1, 1), lambda i: (0, 0)),
    pl.BlockSpec((1024, C), lambda i: (i, 0)),
    pl.BlockSpec((1, C), lambda i: (0, 0)),
]


def _tc_node_layer(acc0, acc1, den0, den1, deg0, deg1, sg0, sg1, av, dv, kv,
                   xp, b, W, as_, ad_):
    BN = 1024

    def body(a0, a1, dn0, dn1, dg0, dg1, s0, s1, a_r, d_r, k_r, xp_r, b_r,
             w_ref, as_ref, ad_ref, xp2_ref, a_ref, d_ref):
        h = _combine(dn0, dn1, dg0, dg1, s0, s1, a_r, d_r, k_r, a0, a1,
                     xp_r, b_r)
        xp2 = _dot(h, w_ref[...])
        xp2_ref[...] = xp2
        a_ref[...] = _dot(xp2, as_ref[...])
        d_ref[...] = _dot(xp2, ad_ref[...])

    return pl.pallas_call(
        body,
        grid=(NP // BN,),
        in_specs=_NODE_SPECS + [
            pl.BlockSpec((C, C), lambda i: (0, 0)),
            pl.BlockSpec((C, 1), lambda i: (0, 0)),
            pl.BlockSpec((C, 1), lambda i: (0, 0)),
        ],
        out_specs=[
            pl.BlockSpec((BN, C), lambda i: (i, 0)),
            pl.BlockSpec((BN, 1), lambda i: (i, 0)),
            pl.BlockSpec((BN, 1), lambda i: (i, 0)),
        ],
        out_shape=[
            jax.ShapeDtypeStruct((NP, C), jnp.float32),
            jax.ShapeDtypeStruct((NP, 1), jnp.float32),
            jax.ShapeDtypeStruct((NP, 1), jnp.float32),
        ],
    )(acc0, acc1, den0, den1, deg0, deg1, sg0, sg1, av, dv, kv, xp,
      b.reshape(1, C), W, as_.reshape(C, 1), ad_.reshape(C, 1))


def _tc_node_mid(acc0, acc1, den0, den1, deg0, deg1, sg0, sg1, av, dv, kv,
                 xp, b, Wa, Wb):
    BN = 1024

    def body(a0, a1, dn0, dn1, dg0, dg1, s0, s1, a_r, d_r, k_r, xp_r, b_r,
             wa_ref, wb_ref, oa_ref, ob_ref):
        h = _combine(dn0, dn1, dg0, dg1, s0, s1, a_r, d_r, k_r, a0, a1,
                     xp_r, b_r)
        oa_ref[...] = _dot(h, wa_ref[...])
        ob_ref[...] = _dot(h, wb_ref[...])

    return pl.pallas_call(
        body,
        grid=(NP // BN,),
        in_specs=_NODE_SPECS + [
            pl.BlockSpec((C, C), lambda i: (0, 0)),
            pl.BlockSpec((C, C), lambda i: (0, 0)),
        ],
        out_specs=[
            pl.BlockSpec((BN, C), lambda i: (i, 0)),
            pl.BlockSpec((BN, C), lambda i: (i, 0)),
        ],
        out_shape=[
            jax.ShapeDtypeStruct((NP, C), jnp.float32),
            jax.ShapeDtypeStruct((NP, C), jnp.float32),
        ],
    )(acc0, acc1, den0, den1, deg0, deg1, sg0, sg1, av, dv, kv, xp,
      b.reshape(1, C), Wa, Wb)


def _sc_layer1(aT, dT, g1F, g2F, srcF, dstF, kv, xp):
    @functools.partial(
        pl.kernel,
        out_type=(
            jax.ShapeDtypeStruct((NCORE * NP, C), jnp.float32),
            jax.ShapeDtypeStruct((NCORE * NP,), jnp.float32),
            jax.ShapeDtypeStruct((NCORE * NP,), jnp.float32),
            jax.ShapeDtypeStruct((NCORE * NP,), jnp.float32),
            jax.ShapeDtypeStruct((NCORE * NP,), jnp.float32),
        ),
        mesh=_MESH,
        compiler_params=_SC_PARAMS,
        scratch_types=[
            pltpu.VMEM((NP,), jnp.float32),
            pltpu.VMEM((NP,), jnp.float32),
            pltpu.VMEM((16,), jnp.float32),
            pltpu.VMEM((EPWP,), jnp.int32),
            pltpu.VMEM((EPWP,), jnp.int32),
            pltpu.VMEM((EPWP,), jnp.float32),
            pltpu.VMEM((EPWP,), jnp.float32),
            pltpu.VMEM((EPWP,), jnp.float32),
            pltpu.VMEM((NPS,), jnp.float32),
            pltpu.VMEM((128, C), jnp.float32),
            pltpu.VMEM_SHARED((NP,), jnp.float32),
            pltpu.VMEM_SHARED((NP,), jnp.float32),
            pltpu.VMEM_SHARED((NP,), jnp.float32),
            pltpu.VMEM_SHARED((NP,), jnp.float32),
            pltpu.VMEM_SHARED((NP, C), jnp.float32),
        ],
    )
    def k(aT_h, dT_h, g1_h, g2_h, srcF_h, dstF_h, kv_h, xp_h,
          acc_o, den_o, deg_o, sg1_o, sg2_o,
          ta, td, kvv, srcf, dstf, g1v, g2v, exv, zv,
          rows, dens, degs, sg1s, sg2s, accs):
        c = lax.axis_index("c")
        s = lax.axis_index("s")
        w = c * NSUB + s

        @pl.loop(0, NPS, step=16)
        def _(i):
            zv[pl.ds(i, 16)] = jnp.zeros((16,), jnp.float32)

        @pl.loop(0, 128)
        def _(r):
            @pl.loop(0, C, step=16)
            def _(i):
                rows[r, pl.ds(i, 16)] = jnp.zeros((16,), jnp.float32)

        ns = s * NPS
        pltpu.sync_copy(zv, dens.at[pl.ds(ns, NPS)])
        pltpu.sync_copy(zv, degs.at[pl.ds(ns, NPS)])
        pltpu.sync_copy(zv, sg1s.at[pl.ds(ns, NPS)])
        pltpu.sync_copy(zv, sg2s.at[pl.ds(ns, NPS)])

        @pl.loop(0, NPS, step=128)
        def _(i):
            pltpu.sync_copy(rows, accs.at[pl.ds(ns + i, 128)])

        eb = w * EPWP
        esl = pl.ds(eb, EPWP)
        pltpu.sync_copy(aT_h, ta)
        pltpu.sync_copy(dT_h, td)
        pltpu.sync_copy(kv_h, kvv)
        pltpu.sync_copy(srcF_h.at[esl], srcf)
        pltpu.sync_copy(dstF_h.at[esl], dstf)
        pltpu.sync_copy(g1_h.at[esl], g1v)
        pltpu.sync_copy(g2_h.at[esl], g2v)
        plsc.subcore_barrier()
        K = kvv[...]

        @pl.loop(0, EPWP, step=16)
        def _(i):
            g = pl.ds(i, 16)
            av = plsc.load_gather(ta, [srcf[g]])
            dv = plsc.load_gather(td, [dstf[g]])
            al = _lrelu(av + dv + g1v[g])
            exv[g] = jnp.exp(al - K)

        pltpu.sync_copy(exv, dens.at[dstf], add=True)
        pltpu.sync_copy(g1v, sg1s.at[dstf], add=True)
        pltpu.sync_copy(g2v, sg2s.at[dstf], add=True)

        @pl.loop(0, EPWP, step=16)
        def _(i):
            g2v[pl.ds(i, 16)] = jnp.ones((16,), jnp.float32)

        pltpu.sync_copy(g2v, degs.at[dstf], add=True)

        @pl.loop(0, RPW)
        def _(j):
            pltpu.sync_copy(xp_h.at[srcf.at[pl.ds(j * 128, 128)]], rows)

            @pl.loop(0, 128)
            def _(e):
                esplat = jnp.zeros((16,), jnp.int32) + (j * 128 + e)
                we = plsc.load_gather(exv, [esplat])
                for cb in range(C // 16):
                    g = pl.ds(cb * 16, 16)
                    rows[e, g] = rows[e, g] * we

            pltpu.sync_copy(rows, accs.at[dstf.at[pl.ds(j * 128, 128)]],
                            add=True)

        plsc.subcore_barrier()
        sl = pl.ds(ns, NPS)
        osl = pl.ds(c * NP + ns, NPS)
        pltpu.sync_copy(accs.at[sl], acc_o.at[osl])
        pltpu.sync_copy(dens.at[sl], den_o.at[osl])
        pltpu.sync_copy(degs.at[sl], deg_o.at[osl])
        pltpu.sync_copy(sg1s.at[sl], sg1_o.at[osl])
        pltpu.sync_copy(sg2s.at[sl], sg2_o.at[osl])

    return k(aT, dT, g1F, g2F, srcF, dstF, kv, xp)


def _sc_layer2(aT, dT, gF, srcF, dstF, kv, xp):
    @functools.partial(
        pl.kernel,
        out_type=(
            jax.ShapeDtypeStruct((NCORE * NP, C), jnp.float32),
            jax.ShapeDtypeStruct((NCORE * NP,), jnp.float32),
        ),
        mesh=_MESH,
        compiler_params=_SC_PARAMS,
        scratch_types=[
            pltpu.VMEM((NP,), jnp.float32),
            pltpu.VMEM((NP,), jnp.float32),
            pltpu.VMEM((16,), jnp.float32),
            pltpu.VMEM((EPWP,), jnp.int32),
            pltpu.VMEM((EPWP,), jnp.int32),
            pltpu.VMEM((EPWP,), jnp.float32),
            pltpu.VMEM((EPWP,), jnp.float32),
            pltpu.VMEM((NPS,), jnp.float32),
            pltpu.VMEM((128, C), jnp.float32),
            pltpu.VMEM_SHARED((NP,), jnp.float32),
            pltpu.VMEM_SHARED((NP, C), jnp.float32),
        ],
    )
    def k(aT_h, dT_h, g_h, srcF_h, dstF_h, kv_h, xp_h,
          acc_o, den_o,
          ta, td, kvv, srcf, dstf, gv, exv, zv,
          rows, dens, accs):
        c = lax.axis_index("c")
        s = lax.axis_index("s")
        w = c * NSUB + s

        @pl.loop(0, NPS, step=16)
        def _(i):
            zv[pl.ds(i, 16)] = jnp.zeros((16,), jnp.float32)

        @pl.loop(0, 128)
        def _(r):
            @pl.loop(0, C, step=16)
            def _(i):
                rows[r, pl.ds(i, 16)] = jnp.zeros((16,), jnp.float32)

        ns = s * NPS
        pltpu.sync_copy(zv, dens.at[pl.ds(ns, NPS)])

        @pl.loop(0, NPS, step=128)
        def _(i):
            pltpu.sync_copy(rows, accs.at[pl.ds(ns + i, 128)])

        eb = w * EPWP
        esl = pl.ds(eb, EPWP)
        pltpu.sync_copy(aT_h, ta)
        pltpu.sync_copy(dT_h, td)
        pltpu.sync_copy(kv_h, kvv)
        pltpu.sync_copy(srcF_h.at[esl], srcf)
        pltpu.sync_copy(dstF_h.at[esl], dstf)
        pltpu.sync_copy(g_h.at[esl], gv)
        plsc.subcore_barrier()
        K = kvv[...]

        @pl.loop(0, EPWP, step=16)
        def _(i):
            g = pl.ds(i, 16)
            av = plsc.load_gather(ta, [srcf[g]])
            dv = plsc.load_gather(td, [dstf[g]])
            al = _lrelu(av + dv + gv[g])
            exv[g] = jnp.exp(al - K)

        pltpu.sync_copy(exv, dens.at[dstf], add=True)

        @pl.loop(0, RPW)
        def _(j):
            pltpu.sync_copy(xp_h.at[srcf.at[pl.ds(j * 128, 128)]], rows)

            @pl.loop(0, 128)
            def _(e):
                esplat = jnp.zeros((16,), jnp.int32) + (j * 128 + e)
                we = plsc.load_gather(exv, [esplat])
                for cb in range(C // 16):
                    g = pl.ds(cb * 16, 16)
                    rows[e, g] = rows[e, g] * we

            pltpu.sync_copy(rows, accs.at[dstf.at[pl.ds(j * 128, 128)]],
                            add=True)

        plsc.subcore_barrier()
        sl = pl.ds(ns, NPS)
        osl = pl.ds(c * NP + ns, NPS)
        pltpu.sync_copy(accs.at[sl], acc_o.at[osl])
        pltpu.sync_copy(dens.at[sl], den_o.at[osl])

    return k(aT, dT, gF, srcF, dstF, kv, xp)


GB = 512


def _sc_final(Pt, Qt, RP, srcF, dstF, wm2, b2v):
    @functools.partial(
        pl.kernel,
        out_type=jax.ShapeDtypeStruct((EP,), jnp.float32),
        mesh=_MESH,
        compiler_params=_SC_PARAMS,
        scratch_types=[
            pltpu.VMEM((EPWP,), jnp.int32),
            pltpu.VMEM((EPWP,), jnp.int32),
            pltpu.VMEM((GB, C), jnp.float32),
            pltpu.VMEM((GB, C), jnp.float32),
            pltpu.VMEM((GB, C), jnp.float32),
            pltpu.VMEM((C,), jnp.float32),
            pltpu.VMEM((16,), jnp.float32),
            pltpu.VMEM((16, 16), jnp.float32),
            pltpu.VMEM((EPWP,), jnp.float32),
        ],
    )
    def k(p_h, q_h, r_h, src_h, dst_h, wm2_h, b2_h,
          out_o,
          srcv, dstv, prow, qrow, rrow, tw, bv, part, outv):
        c = lax.axis_index("c")
        s = lax.axis_index("s")
        w = c * NSUB + s
        pltpu.sync_copy(wm2_h, tw)
        pltpu.sync_copy(b2_h, bv)
        eb = w * EPWP
        pltpu.sync_copy(src_h.at[pl.ds(eb, EPWP)], srcv)
        pltpu.sync_copy(dst_h.at[pl.ds(eb, EPWP)], dstv)
        m0 = tw[pl.ds(0, 16)]
        m1 = tw[pl.ds(16, 16)]
        m2 = tw[pl.ds(32, 16)]
        m3 = tw[pl.ds(48, 16)]
        bias = bv[...]
        riota = lax.iota(jnp.int32, 16)

        @pl.loop(0, EPWP, step=GB)
        def _(b):
            pltpu.sync_copy(p_h.at[srcv.at[pl.ds(b, GB)]], prow)
            pltpu.sync_copy(q_h.at[dstv.at[pl.ds(b, GB)]], qrow)
            pltpu.sync_copy(r_h.at[pl.ds(eb + b, GB)], rrow)

            @pl.loop(0, GB, step=16)
            def _(i):
                @pl.loop(0, 16)
                def _(e2):
                    e = i + e2
                    g0 = pl.ds(0, 16)
                    g1 = pl.ds(16, 16)
                    g2 = pl.ds(32, 16)
                    g3 = pl.ds(48, 16)
                    t0 = jnp.maximum(prow[e, g0] + qrow[e, g0] + rrow[e, g0], 0.0)
                    t1 = jnp.maximum(prow[e, g1] + qrow[e, g1] + rrow[e, g1], 0.0)
                    t2 = jnp.maximum(prow[e, g2] + qrow[e, g2] + rrow[e, g2], 0.0)
                    t3 = jnp.maximum(prow[e, g3] + qrow[e, g3] + rrow[e, g3], 0.0)
                    part[e2, :] = t0 * m0 + t1 * m1 + t2 * m2 + t3 * m3

                acc = bias

                def col(l, a):
                    cv = plsc.load_gather(part, [riota, jnp.full((16,), l, jnp.int32)])
                    return a + cv

                acc = lax.fori_loop(0, 16, col, acc)
                outv[pl.ds(b + i, 16)] = acc

        pltpu.sync_copy(outv, out_o.at[pl.ds(eb, EPWP)])

    return k(Pt, Qt, RP, srcF, dstF, wm2, b2v)


def kernel(x, edge_index, edge_attr, W1, as1, ad1, We1, ae1, b1,
           W2, as2, ad2, We2, ae2, b2, Wm1, bm1, Wm2, bm2):
    src = edge_index[0]
    dst = edge_index[1]

    xpad = jnp.pad(x, ((0, NP - N), (0, 0)))
    srcF = jnp.pad(src.reshape(NW, EPW), ((0, 0), (0, EPWP - EPW)),
                   constant_values=0).reshape(EP)
    dstF = jnp.pad(dst.reshape(NW, EPW), ((0, 0), (0, EPWP - EPW)),
                   constant_values=PAD_NODE).reshape(EP)
    srcP = srcF.reshape(ROWS, 128)
    dstP = dstF.reshape(ROWS, 128)
    eaP = jnp.pad(edge_attr.reshape(NW, EPW, ED), ((0, 0), (0, EPWP - EPW), (0, 0))
                  ).reshape(EP, ED)

    xp1, a1, d1 = _tc_node_pre(xpad, W1, as1, ad1)
    g1, g2, RP = _tc_edge(eaP, We1, ae1, We2, ae2, Wm1[C:C + ED], bm1)
    a1f = a1.reshape(NP)
    d1f = d1.reshape(NP)
    g1F = g1.reshape(EP)
    g2F = g2.reshape(EP)

    K1 = _lrelu(jnp.max(a1f) + jnp.max(d1f) + jnp.maximum(jnp.max(g1F), 0.0))
    kv1 = jnp.full((16,), K1, jnp.float32)

    accf1, den1p, degp, sg1p, sg2p = _sc_layer1(
        a1f, d1f, g1F, g2F, srcF, dstF, kv1, xp1)

    xp2, a2, d2 = _tc_node_layer(
        accf1[:NP], accf1[NP:],
        den1p[:NP].reshape(NP, 1), den1p[NP:].reshape(NP, 1),
        degp[:NP].reshape(NP, 1), degp[NP:].reshape(NP, 1),
        sg1p[:NP].reshape(NP, 1), sg1p[NP:].reshape(NP, 1),
        a1, d1, K1.reshape(1, 1), xp1, b1, W2, as2, ad2)
    a2f = a2.reshape(NP)
    d2f = d2.reshape(NP)
    K2 = _lrelu(jnp.max(a2f) + jnp.max(d2f) + jnp.maximum(jnp.max(g2F), 0.0))
    kv2 = jnp.full((16,), K2, jnp.float32)

    accf2, den2p = _sc_layer2(a2f, d2f, g2F, srcF, dstF, kv2, xp2)

    Pt, Qt = _tc_node_mid(
        accf2[:NP], accf2[NP:],
        den2p[:NP].reshape(NP, 1), den2p[NP:].reshape(NP, 1),
        degp[:NP].reshape(NP, 1), degp[NP:].reshape(NP, 1),
        sg2p[:NP].reshape(NP, 1), sg2p[NP:].reshape(NP, 1),
        a2, d2, K2.reshape(1, 1), xp2, b2, Wm1[:C], Wm1[C + ED:])

    b2v = jnp.full((16,), bm2[0], jnp.float32)
    outP = _sc_final(Pt, Qt, RP, srcF, dstF, Wm2.reshape(C), b2v)

    out = outP.reshape(NW, EPWP)[:, :EPW].reshape(E, 1)
    return out


# speedup vs baseline: 1.1747x; 1.0084x over previous
import functools

import jax
import jax.numpy as jnp
from jax import lax
from jax.experimental import pallas as pl
from jax.experimental.pallas import tpu as pltpu
from jax.experimental.pallas import tpu_sc as plsc

N = 10000
E = 320000
D = 128
C = 64
ED = 16

NP = 10240
NCORE = 2
NSUB = 16
NW = NCORE * NSUB
NPS = NP // NSUB
EPW = E // NW
EPWP = 10240
EP = NW * EPWP
ROWS = EP // 128
RPW = ROWS // NW
PAD_NODE = NP - 1
GB = 512
GBL = 160

_MESH = plsc.VectorSubcoreMesh(core_axis_name="c", subcore_axis_name="s",
                               num_cores=NCORE, num_subcores=NSUB)

_SC_PARAMS = pltpu.CompilerParams(needs_layout_passes=False,
                                  use_tc_tiling_on_sc=False)

_HIGH = jax.lax.Precision.HIGHEST


def _dot(a, b):
    return jax.lax.dot_general(a, b, (((1,), (0,)), ((), ())),
                               precision=_HIGH,
                               preferred_element_type=jnp.float32)


def _lrelu(x):
    return jnp.maximum(x, 0.2 * x)


def _tc_node_pre(xpad, W1, as1, ad1):
    BN = 1024

    def body(x_ref, w_ref, as_ref, ad_ref, xp_ref, a_ref, d_ref):
        xp = _dot(x_ref[...], w_ref[...])
        xp_ref[...] = xp
        a_ref[...] = _dot(xp, as_ref[...])
        d_ref[...] = _dot(xp, ad_ref[...])

    return pl.pallas_call(
        body,
        grid=(NP // BN,),
        in_specs=[
            pl.BlockSpec((BN, D), lambda i: (i, 0)),
            pl.BlockSpec((D, C), lambda i: (0, 0)),
            pl.BlockSpec((C, 1), lambda i: (0, 0)),
            pl.BlockSpec((C, 1), lambda i: (0, 0)),
        ],
        out_specs=[
            pl.BlockSpec((BN, C), lambda i: (i, 0)),
            pl.BlockSpec((BN, 1), lambda i: (i, 0)),
            pl.BlockSpec((BN, 1), lambda i: (i, 0)),
        ],
        out_shape=[
            jax.ShapeDtypeStruct((NP, C), jnp.float32),
            jax.ShapeDtypeStruct((NP, 1), jnp.float32),
            jax.ShapeDtypeStruct((NP, 1), jnp.float32),
        ],
    )(xpad, W1, as1.reshape(C, 1), ad1.reshape(C, 1))


def _tc_edge(eaP, We1, ae1, We2, ae2, Wm1b, bm1):
    BE = 8192

    def body(ea_ref, we1_ref, ae1_ref, we2_ref, ae2_ref, wmb_ref, bm1_ref,
             g1_ref, g2_ref, r_ref):
        ea = ea_ref[...]
        v1 = _dot(we1_ref[...], ae1_ref[...])
        v2 = _dot(we2_ref[...], ae2_ref[...])
        g1_ref[...] = _dot(ea, v1)
        g2_ref[...] = _dot(ea, v2)
        r_ref[...] = _dot(ea, wmb_ref[...]) + bm1_ref[...]

    return pl.pallas_call(
        body,
        grid=(EP // BE,),
        in_specs=[
            pl.BlockSpec((BE, ED), lambda i: (i, 0)),
            pl.BlockSpec((ED, C), lambda i: (0, 0)),
            pl.BlockSpec((C, 1), lambda i: (0, 0)),
            pl.BlockSpec((ED, C), lambda i: (0, 0)),
            pl.BlockSpec((C, 1), lambda i: (0, 0)),
            pl.BlockSpec((ED, C), lambda i: (0, 0)),
            pl.BlockSpec((1, C), lambda i: (0, 0)),
        ],
        out_specs=[
            pl.BlockSpec((BE, 1), lambda i: (i, 0)),
            pl.BlockSpec((BE, 1), lambda i: (i, 0)),
            pl.BlockSpec((BE, C), lambda i: (i, 0)),
        ],
        out_shape=[
            jax.ShapeDtypeStruct((EP, 1), jnp.float32),
            jax.ShapeDtypeStruct((EP, 1), jnp.float32),
            jax.ShapeDtypeStruct((EP, C), jnp.float32),
        ],
    )(eaP, We1, ae1.reshape(C, 1), We2, ae2.reshape(C, 1), Wm1b,
      bm1.reshape(1, C))


def _combine(dn0, dn1, dg0, dg1, s0, s1, a_r, d_r, k_r, a0, a1, xp_r, b_r):
    deg = jnp.maximum(dg0[...] + dg1[...], 1.0)
    gs = (s0[...] + s1[...]) / deg
    al = _lrelu(a_r[...] + d_r[...] + gs)
    exs = jnp.exp(al - k_r[...])
    den = dn0[...] + dn1[...] + exs
    h = (a0[...] + a1[...] + exs * xp_r[...]) / den + b_r[...]
    return jnp.maximum(h, 0.0)


_NODE_SPECS = [
    pl.BlockSpec((1024, C), lambda i: (i, 0)),
    pl.BlockSpec((1024, C), lambda i: (i, 0)),
    pl.BlockSpec((1024, 1), lambda i: (i, 0)),
    pl.BlockSpec((1024, 1), lambda i: (i, 0)),
    pl.BlockSpec((1024, 1), lambda i: (i, 0)),
    pl.BlockSpec((1024, 1), lambda i: (i, 0)),
    pl.BlockSpec((1024, 1), lambda i: (i, 0)),
    pl.BlockSpec((1024, 1), lambda i: (i, 0)),
    pl.BlockSpec((1024, 1), lambda i: (i, 0)),
    pl.BlockSpec((1024, 1), lambda i: (i, 0)),
    pl.BlockSpec((1, 1), lambda i: (0, 0)),
    pl.BlockSpec((1024, C), lambda i: (i, 0)),
    pl.BlockSpec((1, C), lambda i: (0, 0)),
]


def _tc_node_layer(acc0, acc1, den0, den1, deg0, deg1, sg0, sg1, av, dv, kv,
                   xp, b, W, as_, ad_):
    BN = 1024

    def body(a0, a1, dn0, dn1, dg0, dg1, s0, s1, a_r, d_r, k_r, xp_r, b_r,
             w_ref, as_ref, ad_ref, xp2_ref, a_ref, d_ref):
        h = _combine(dn0, dn1, dg0, dg1, s0, s1, a_r, d_r, k_r, a0, a1,
                     xp_r, b_r)
        xp2 = _dot(h, w_ref[...])
        xp2_ref[...] = xp2
        a_ref[...] = _dot(xp2, as_ref[...])
        d_ref[...] = _dot(xp2, ad_ref[...])

    return pl.pallas_call(
        body,
        grid=(NP // BN,),
        in_specs=_NODE_SPECS + [
            pl.BlockSpec((C, C), lambda i: (0, 0)),
            pl.BlockSpec((C, 1), lambda i: (0, 0)),
            pl.BlockSpec((C, 1), lambda i: (0, 0)),
        ],
        out_specs=[
            pl.BlockSpec((BN, C), lambda i: (i, 0)),
            pl.BlockSpec((BN, 1), lambda i: (i, 0)),
            pl.BlockSpec((BN, 1), lambda i: (i, 0)),
        ],
        out_shape=[
            jax.ShapeDtypeStruct((NP, C), jnp.float32),
            jax.ShapeDtypeStruct((NP, 1), jnp.float32),
            jax.ShapeDtypeStruct((NP, 1), jnp.float32),
        ],
    )(acc0, acc1, den0, den1, deg0, deg1, sg0, sg1, av, dv, kv, xp,
      b.reshape(1, C), W, as_.reshape(C, 1), ad_.reshape(C, 1))


def _tc_node_mid(acc0, acc1, den0, den1, deg0, deg1, sg0, sg1, av, dv, kv,
                 xp, b, Wa, Wb):
    BN = 1024

    def body(a0, a1, dn0, dn1, dg0, dg1, s0, s1, a_r, d_r, k_r, xp_r, b_r,
             wa_ref, wb_ref, oa_ref, ob_ref):
        h = _combine(dn0, dn1, dg0, dg1, s0, s1, a_r, d_r, k_r, a0, a1,
                     xp_r, b_r)
        oa_ref[...] = _dot(h, wa_ref[...])
        ob_ref[...] = _dot(h, wb_ref[...])

    return pl.pallas_call(
        body,
        grid=(NP // BN,),
        in_specs=_NODE_SPECS + [
            pl.BlockSpec((C, C), lambda i: (0, 0)),
            pl.BlockSpec((C, C), lambda i: (0, 0)),
        ],
        out_specs=[
            pl.BlockSpec((BN, C), lambda i: (i, 0)),
            pl.BlockSpec((BN, C), lambda i: (i, 0)),
        ],
        out_shape=[
            jax.ShapeDtypeStruct((NP, C), jnp.float32),
            jax.ShapeDtypeStruct((NP, C), jnp.float32),
        ],
    )(acc0, acc1, den0, den1, deg0, deg1, sg0, sg1, av, dv, kv, xp,
      b.reshape(1, C), Wa, Wb)


def _sc_layer1(aT, dT, g1F, g2F, srcF, dstF, kv, xp):
    @functools.partial(
        pl.kernel,
        out_type=(
            jax.ShapeDtypeStruct((NCORE * NP, C), jnp.float32),
            jax.ShapeDtypeStruct((NCORE * NP,), jnp.float32),
            jax.ShapeDtypeStruct((NCORE * NP,), jnp.float32),
            jax.ShapeDtypeStruct((NCORE * NP,), jnp.float32),
            jax.ShapeDtypeStruct((NCORE * NP,), jnp.float32),
        ),
        mesh=_MESH,
        compiler_params=_SC_PARAMS,
        scratch_types=[
            pltpu.VMEM((NP,), jnp.float32),
            pltpu.VMEM((NP,), jnp.float32),
            pltpu.VMEM((16,), jnp.float32),
            pltpu.VMEM((EPWP,), jnp.int32),
            pltpu.VMEM((EPWP,), jnp.int32),
            pltpu.VMEM((EPWP,), jnp.float32),
            pltpu.VMEM((EPWP,), jnp.float32),
            pltpu.VMEM((EPWP,), jnp.float32),
            pltpu.VMEM((NPS,), jnp.float32),
            pltpu.VMEM((GBL, C), jnp.float32),
            pltpu.VMEM_SHARED((NP,), jnp.float32),
            pltpu.VMEM_SHARED((NP,), jnp.float32),
            pltpu.VMEM_SHARED((NP,), jnp.float32),
            pltpu.VMEM_SHARED((NP,), jnp.float32),
            pltpu.VMEM_SHARED((NP, C), jnp.float32),
        ],
    )
    def k(aT_h, dT_h, g1_h, g2_h, srcF_h, dstF_h, kv_h, xp_h,
          acc_o, den_o, deg_o, sg1_o, sg2_o,
          ta, td, kvv, srcf, dstf, g1v, g2v, exv, zv,
          rows, dens, degs, sg1s, sg2s, accs):
        c = lax.axis_index("c")
        s = lax.axis_index("s")
        w = c * NSUB + s

        @pl.loop(0, NPS, step=16)
        def _(i):
            zv[pl.ds(i, 16)] = jnp.zeros((16,), jnp.float32)

        @pl.loop(0, 128)
        def _(r):
            @pl.loop(0, C, step=16)
            def _(i):
                rows[r, pl.ds(i, 16)] = jnp.zeros((16,), jnp.float32)

        ns = s * NPS
        pltpu.sync_copy(zv, dens.at[pl.ds(ns, NPS)])
        pltpu.sync_copy(zv, degs.at[pl.ds(ns, NPS)])
        pltpu.sync_copy(zv, sg1s.at[pl.ds(ns, NPS)])
        pltpu.sync_copy(zv, sg2s.at[pl.ds(ns, NPS)])

        @pl.loop(0, NPS, step=128)
        def _(i):
            pltpu.sync_copy(rows.at[pl.ds(0, 128)], accs.at[pl.ds(ns + i, 128)])

        eb = w * EPWP
        esl = pl.ds(eb, EPWP)
        pltpu.sync_copy(aT_h, ta)
        pltpu.sync_copy(dT_h, td)
        pltpu.sync_copy(kv_h, kvv)
        pltpu.sync_copy(srcF_h.at[esl], srcf)
        pltpu.sync_copy(dstF_h.at[esl], dstf)
        pltpu.sync_copy(g1_h.at[esl], g1v)
        pltpu.sync_copy(g2_h.at[esl], g2v)
        plsc.subcore_barrier()
        K = kvv[...]

        @pl.loop(0, EPWP, step=16)
        def _(i):
            g = pl.ds(i, 16)
            av = plsc.load_gather(ta, [srcf[g]])
            dv = plsc.load_gather(td, [dstf[g]])
            al = _lrelu(av + dv + g1v[g])
            exv[g] = jnp.exp(al - K)

        pltpu.sync_copy(exv, dens.at[dstf], add=True)
        pltpu.sync_copy(g1v, sg1s.at[dstf], add=True)
        pltpu.sync_copy(g2v, sg2s.at[dstf], add=True)

        @pl.loop(0, EPWP, step=16)
        def _(i):
            g2v[pl.ds(i, 16)] = jnp.ones((16,), jnp.float32)

        pltpu.sync_copy(g2v, degs.at[dstf], add=True)

        @pl.loop(0, EPWP, step=GBL)
        def _(b):
            pltpu.sync_copy(xp_h.at[srcf.at[pl.ds(b, GBL)]], rows)

            @pl.loop(0, GBL)
            def _(e):
                esplat = jnp.zeros((16,), jnp.int32) + (b + e)
                we = plsc.load_gather(exv, [esplat])
                for cb in range(C // 16):
                    g = pl.ds(cb * 16, 16)
                    rows[e, g] = rows[e, g] * we

            pltpu.sync_copy(rows, accs.at[dstf.at[pl.ds(b, GBL)]],
                            add=True)

        plsc.subcore_barrier()
        sl = pl.ds(ns, NPS)
        osl = pl.ds(c * NP + ns, NPS)
        pltpu.sync_copy(accs.at[sl], acc_o.at[osl])
        pltpu.sync_copy(dens.at[sl], den_o.at[osl])
        pltpu.sync_copy(degs.at[sl], deg_o.at[osl])
        pltpu.sync_copy(sg1s.at[sl], sg1_o.at[osl])
        pltpu.sync_copy(sg2s.at[sl], sg2_o.at[osl])

    return k(aT, dT, g1F, g2F, srcF, dstF, kv, xp)


def _sc_layer2(aT, dT, gF, srcF, dstF, kv, xp):
    @functools.partial(
        pl.kernel,
        out_type=(
            jax.ShapeDtypeStruct((NCORE * NP, C), jnp.float32),
            jax.ShapeDtypeStruct((NCORE * NP,), jnp.float32),
        ),
        mesh=_MESH,
        compiler_params=_SC_PARAMS,
        scratch_types=[
            pltpu.VMEM((NP,), jnp.float32),
            pltpu.VMEM((NP,), jnp.float32),
            pltpu.VMEM((16,), jnp.float32),
            pltpu.VMEM((EPWP,), jnp.int32),
            pltpu.VMEM((EPWP,), jnp.int32),
            pltpu.VMEM((EPWP,), jnp.float32),
            pltpu.VMEM((EPWP,), jnp.float32),
            pltpu.VMEM((NPS,), jnp.float32),
            pltpu.VMEM((GBL, C), jnp.float32),
            pltpu.VMEM_SHARED((NP,), jnp.float32),
            pltpu.VMEM_SHARED((NP, C), jnp.float32),
        ],
    )
    def k(aT_h, dT_h, g_h, srcF_h, dstF_h, kv_h, xp_h,
          acc_o, den_o,
          ta, td, kvv, srcf, dstf, gv, exv, zv,
          rows, dens, accs):
        c = lax.axis_index("c")
        s = lax.axis_index("s")
        w = c * NSUB + s

        @pl.loop(0, NPS, step=16)
        def _(i):
            zv[pl.ds(i, 16)] = jnp.zeros((16,), jnp.float32)

        @pl.loop(0, 128)
        def _(r):
            @pl.loop(0, C, step=16)
            def _(i):
                rows[r, pl.ds(i, 16)] = jnp.zeros((16,), jnp.float32)

        ns = s * NPS
        pltpu.sync_copy(zv, dens.at[pl.ds(ns, NPS)])

        @pl.loop(0, NPS, step=128)
        def _(i):
            pltpu.sync_copy(rows.at[pl.ds(0, 128)], accs.at[pl.ds(ns + i, 128)])

        eb = w * EPWP
        esl = pl.ds(eb, EPWP)
        pltpu.sync_copy(aT_h, ta)
        pltpu.sync_copy(dT_h, td)
        pltpu.sync_copy(kv_h, kvv)
        pltpu.sync_copy(srcF_h.at[esl], srcf)
        pltpu.sync_copy(dstF_h.at[esl], dstf)
        pltpu.sync_copy(g_h.at[esl], gv)
        plsc.subcore_barrier()
        K = kvv[...]

        @pl.loop(0, EPWP, step=16)
        def _(i):
            g = pl.ds(i, 16)
            av = plsc.load_gather(ta, [srcf[g]])
            dv = plsc.load_gather(td, [dstf[g]])
            al = _lrelu(av + dv + gv[g])
            exv[g] = jnp.exp(al - K)

        pltpu.sync_copy(exv, dens.at[dstf], add=True)

        @pl.loop(0, EPWP, step=GBL)
        def _(b):
            pltpu.sync_copy(xp_h.at[srcf.at[pl.ds(b, GBL)]], rows)

            @pl.loop(0, GBL)
            def _(e):
                esplat = jnp.zeros((16,), jnp.int32) + (b + e)
                we = plsc.load_gather(exv, [esplat])
                for cb in range(C // 16):
                    g = pl.ds(cb * 16, 16)
                    rows[e, g] = rows[e, g] * we

            pltpu.sync_copy(rows, accs.at[dstf.at[pl.ds(b, GBL)]],
                            add=True)

        plsc.subcore_barrier()
        sl = pl.ds(ns, NPS)
        osl = pl.ds(c * NP + ns, NPS)
        pltpu.sync_copy(accs.at[sl], acc_o.at[osl])
        pltpu.sync_copy(dens.at[sl], den_o.at[osl])

    return k(aT, dT, gF, srcF, dstF, kv, xp)


def _sc_final(Pt, Qt, RP, srcF, dstF, wm2, b2v):
    @functools.partial(
        pl.kernel,
        out_type=jax.ShapeDtypeStruct((EP,), jnp.float32),
        mesh=_MESH,
        compiler_params=_SC_PARAMS,
        scratch_types=[
            pltpu.VMEM((EPWP,), jnp.int32),
            pltpu.VMEM((EPWP,), jnp.int32),
            pltpu.VMEM((GB, C), jnp.float32),
            pltpu.VMEM((GB, C), jnp.float32),
            pltpu.VMEM((GB, C), jnp.float32),
            pltpu.VMEM((C,), jnp.float32),
            pltpu.VMEM((16,), jnp.float32),
            pltpu.VMEM((16, 16), jnp.float32),
            pltpu.VMEM((EPWP,), jnp.float32),
        ],
    )
    def k(p_h, q_h, r_h, src_h, dst_h, wm2_h, b2_h,
          out_o,
          srcv, dstv, prow, qrow, rrow, tw, bv, part, outv):
        c = lax.axis_index("c")
        s = lax.axis_index("s")
        w = c * NSUB + s
        pltpu.sync_copy(wm2_h, tw)
        pltpu.sync_copy(b2_h, bv)
        eb = w * EPWP
        pltpu.sync_copy(src_h.at[pl.ds(eb, EPWP)], srcv)
        pltpu.sync_copy(dst_h.at[pl.ds(eb, EPWP)], dstv)
        m0 = tw[pl.ds(0, 16)]
        m1 = tw[pl.ds(16, 16)]
        m2 = tw[pl.ds(32, 16)]
        m3 = tw[pl.ds(48, 16)]
        bias = bv[...]
        riota = lax.iota(jnp.int32, 16)

        @pl.loop(0, EPWP, step=GB)
        def _(b):
            pltpu.sync_copy(p_h.at[srcv.at[pl.ds(b, GB)]], prow)
            pltpu.sync_copy(q_h.at[dstv.at[pl.ds(b, GB)]], qrow)
            pltpu.sync_copy(r_h.at[pl.ds(eb + b, GB)], rrow)

            @pl.loop(0, GB, step=16)
            def _(i):
                @pl.loop(0, 16)
                def _(e2):
                    e = i + e2
                    g0 = pl.ds(0, 16)
                    g1 = pl.ds(16, 16)
                    g2 = pl.ds(32, 16)
                    g3 = pl.ds(48, 16)
                    t0 = jnp.maximum(prow[e, g0] + qrow[e, g0] + rrow[e, g0], 0.0)
                    t1 = jnp.maximum(prow[e, g1] + qrow[e, g1] + rrow[e, g1], 0.0)
                    t2 = jnp.maximum(prow[e, g2] + qrow[e, g2] + rrow[e, g2], 0.0)
                    t3 = jnp.maximum(prow[e, g3] + qrow[e, g3] + rrow[e, g3], 0.0)
                    part[e2, :] = t0 * m0 + t1 * m1 + t2 * m2 + t3 * m3

                acc = bias

                def col(l, a):
                    cv = plsc.load_gather(part, [riota, jnp.full((16,), l, jnp.int32)])
                    return a + cv

                acc = lax.fori_loop(0, 16, col, acc)
                outv[pl.ds(b + i, 16)] = acc

        pltpu.sync_copy(outv, out_o.at[pl.ds(eb, EPWP)])

    return k(Pt, Qt, RP, srcF, dstF, wm2, b2v)


def kernel(x, edge_index, edge_attr, W1, as1, ad1, We1, ae1, b1,
           W2, as2, ad2, We2, ae2, b2, Wm1, bm1, Wm2, bm2):
    src = edge_index[0]
    dst = edge_index[1]

    xpad = jnp.pad(x, ((0, NP - N), (0, 0)))
    srcF = jnp.pad(src.reshape(NW, EPW), ((0, 0), (0, EPWP - EPW)),
                   constant_values=0).reshape(EP)
    dstF = jnp.pad(dst.reshape(NW, EPW), ((0, 0), (0, EPWP - EPW)),
                   constant_values=PAD_NODE).reshape(EP)
    srcP = srcF.reshape(ROWS, 128)
    dstP = dstF.reshape(ROWS, 128)
    eaP = jnp.pad(edge_attr.reshape(NW, EPW, ED), ((0, 0), (0, EPWP - EPW), (0, 0))
                  ).reshape(EP, ED)

    xp1, a1, d1 = _tc_node_pre(xpad, W1, as1, ad1)
    g1, g2, RP = _tc_edge(eaP, We1, ae1, We2, ae2, Wm1[C:C + ED], bm1)
    a1f = a1.reshape(NP)
    d1f = d1.reshape(NP)
    g1F = g1.reshape(EP)
    g2F = g2.reshape(EP)

    K1 = _lrelu(jnp.max(a1f) + jnp.max(d1f) + jnp.maximum(jnp.max(g1F), 0.0))
    kv1 = jnp.full((16,), K1, jnp.float32)

    accf1, den1p, degp, sg1p, sg2p = _sc_layer1(
        a1f, d1f, g1F, g2F, srcF, dstF, kv1, xp1)

    xp2, a2, d2 = _tc_node_layer(
        accf1[:NP], accf1[NP:],
        den1p[:NP].reshape(NP, 1), den1p[NP:].reshape(NP, 1),
        degp[:NP].reshape(NP, 1), degp[NP:].reshape(NP, 1),
        sg1p[:NP].reshape(NP, 1), sg1p[NP:].reshape(NP, 1),
        a1, d1, K1.reshape(1, 1), xp1, b1, W2, as2, ad2)
    a2f = a2.reshape(NP)
    d2f = d2.reshape(NP)
    K2 = _lrelu(jnp.max(a2f) + jnp.max(d2f) + jnp.maximum(jnp.max(g2F), 0.0))
    kv2 = jnp.full((16,), K2, jnp.float32)

    accf2, den2p = _sc_layer2(a2f, d2f, g2F, srcF, dstF, kv2, xp2)

    Pt, Qt = _tc_node_mid(
        accf2[:NP], accf2[NP:],
        den2p[:NP].reshape(NP, 1), den2p[NP:].reshape(NP, 1),
        degp[:NP].reshape(NP, 1), degp[NP:].reshape(NP, 1),
        sg2p[:NP].reshape(NP, 1), sg2p[NP:].reshape(NP, 1),
        a2, d2, K2.reshape(1, 1), xp2, b2, Wm1[:C], Wm1[C + ED:])

    b2v = jnp.full((16,), bm2[0], jnp.float32)
    outP = _sc_final(Pt, Qt, RP, srcF, dstF, Wm2.reshape(C), b2v)

    out = outP.reshape(NW, EPWP)[:, :EPW].reshape(E, 1)
    return out
